# Initial kernel scaffold; baseline (speedup 1.0000x reference)
import jax
import jax.numpy as jnp
from jax.experimental import pallas as pl


def kernel(x, edge_index, W1, b1, W2, b2):
    raise NotImplementedError("write your pallas kernel here")


# speedup vs baseline: 16.2322x; 16.2322x over previous
import functools

import jax
import jax.numpy as jnp
from jax.experimental import pallas as pl
from jax.experimental.pallas import tpu as pltpu

N = 10000
D = 128
BLK = 400


def _dense_body(p_ref, nd_ref, w_ref, b_ref, o_ref, *, relu, post_ref=None):
    acc = jnp.dot(p_ref[...] * nd_ref[...], w_ref[...],
                  preferred_element_type=jnp.float32) + b_ref[...]
    if relu:
        acc = jnp.maximum(acc, 0.0)
    if post_ref is not None:
        acc = acc * post_ref[...]
    o_ref[...] = acc


def _dense_stage(agg, nd, w, b, relu, post=None):
    args = [agg, nd, w, b]
    in_specs = [
        pl.BlockSpec((BLK, D), lambda i: (i, jnp.int32(0))),
        pl.BlockSpec((BLK, 1), lambda i: (i, jnp.int32(0))),
        pl.BlockSpec((D, D), lambda i: (jnp.int32(0), jnp.int32(0))),
        pl.BlockSpec((1, D), lambda i: (jnp.int32(0), jnp.int32(0))),
    ]
    body = _dense_body
    if post is not None:
        args.append(post)
        in_specs.append(pl.BlockSpec((BLK, 1), lambda i: (i, jnp.int32(0))))
        body = lambda p, n_, w_, b_, po, o: _dense_body(
            p, n_, w_, b_, o, relu=relu, post_ref=po)
    else:
        body = functools.partial(_dense_body, relu=relu, post_ref=None)
    return pl.pallas_call(
        body,
        grid=(N // BLK,),
        in_specs=in_specs,
        out_specs=pl.BlockSpec((BLK, D), lambda i: (i, 0)),
        out_shape=jax.ShapeDtypeStruct((N, D), jnp.float32),
    )(*args)


def kernel(x, edge_index, W1, b1, W2, b2):
    prev_x64 = jax.config.jax_enable_x64
    jax.config.update("jax_enable_x64", False)
    try:
        return _kernel_impl(x, edge_index, W1, b1, W2, b2)
    finally:
        jax.config.update("jax_enable_x64", prev_x64)


def _kernel_impl(x, edge_index, W1, b1, W2, b2):
    src = edge_index[0].astype(jnp.int32)
    dst = edge_index[1].astype(jnp.int32)
    x = x.astype(jnp.float32)

    deg_out = jnp.zeros((N,), jnp.float32).at[src].add(1.0)
    deg_in = jnp.zeros((N,), jnp.float32).at[dst].add(1.0)
    ns = jnp.where(deg_out > 0, jax.lax.rsqrt(deg_out), 0.0)[:, None]
    nd = jnp.where(deg_in > 0, jax.lax.rsqrt(deg_in), 0.0)[:, None]

    h = x * ns
    agg1 = jnp.zeros((N, D), jnp.float32).at[dst].add(h[src])
    h1s = _dense_stage(agg1, nd, W1, b1.reshape(1, D), relu=True, post=ns)
    agg2 = jnp.zeros((N, D), jnp.float32).at[dst].add(h1s[src])
    out = _dense_stage(agg2, nd, W2, b2.reshape(1, D), relu=False)
    return out


# speedup vs baseline: 75.0678x; 4.6246x over previous
import functools

import jax
import jax.numpy as jnp
from jax import lax
from jax.experimental import pallas as pl
from jax.experimental.pallas import tpu as pltpu
from jax.experimental.pallas import tpu_sc as plsc

N = 10000
NP = 10240
D = 128
DH = 64
E = 320000
EP = 327680
CHUNK = 128
ECHUNKS = EP // CHUNK
ROWS_T = NP // 16
BLK = 512

_MESH = plsc.VectorSubcoreMesh(core_axis_name="c", subcore_axis_name="s",
                               num_cores=2, num_subcores=16)


@functools.partial(
    pl.kernel,
    out_type=(
        jax.ShapeDtypeStruct((2, NP, 16), jnp.float32),
        jax.ShapeDtypeStruct((2, NP, 16), jnp.float32),
    ),
    mesh=_MESH,
    scratch_types=[
        pltpu.VMEM((ECHUNKS // 32, CHUNK), jnp.int32),
        pltpu.VMEM((ECHUNKS // 32, CHUNK), jnp.int32),
        pltpu.VMEM((CHUNK, 16), jnp.float32),
        pltpu.VMEM_SHARED((NP, 16), jnp.float32),
        pltpu.VMEM_SHARED((NP, 16), jnp.float32),
    ],
    compiler_params=pltpu.CompilerParams(use_tc_tiling_on_sc=False),
)
def _deg_kernel(srcp, dstp, ones_h, z16, out_s, out_d,
                si_v, di_v, ones_v, acc_s, acc_d):
    cid = lax.axis_index("c")
    sid = lax.axis_index("s")
    wid = cid * 16 + sid
    nck = ECHUNKS // 32
    pltpu.sync_copy(ones_h, ones_v)
    pltpu.sync_copy(z16, acc_s.at[pl.ds(sid * ROWS_T, ROWS_T)])
    pltpu.sync_copy(z16, acc_d.at[pl.ds(sid * ROWS_T, ROWS_T)])
    pltpu.sync_copy(srcp.at[pl.ds(wid * nck, nck)], si_v)
    pltpu.sync_copy(dstp.at[pl.ds(wid * nck, nck)], di_v)
    plsc.subcore_barrier()

    def body(c, carry):
        pltpu.sync_copy(ones_v, acc_s.at[si_v.at[c]], add=True)
        pltpu.sync_copy(ones_v, acc_d.at[di_v.at[c]], add=True)
        return carry

    lax.fori_loop(0, nck, body, 0)
    plsc.subcore_barrier()
    pltpu.sync_copy(acc_s.at[pl.ds(sid * ROWS_T, ROWS_T)],
                    out_s.at[cid, pl.ds(sid * ROWS_T, ROWS_T)])
    pltpu.sync_copy(acc_d.at[pl.ds(sid * ROWS_T, ROWS_T)],
                    out_d.at[cid, pl.ds(sid * ROWS_T, ROWS_T)])


@functools.partial(
    pl.kernel,
    out_type=jax.ShapeDtypeStruct((2, NP, DH), jnp.float32),
    mesh=_MESH,
    scratch_types=[
        pltpu.VMEM((ECHUNKS // 16, CHUNK), jnp.int32),
        pltpu.VMEM((ECHUNKS // 16, CHUNK), jnp.int32),
        pltpu.VMEM((2, CHUNK, DH), jnp.float32),
        pltpu.VMEM_SHARED((NP, DH), jnp.float32),
        pltpu.SemaphoreType.DMA,
        pltpu.SemaphoreType.DMA,
    ],
    compiler_params=pltpu.CompilerParams(use_tc_tiling_on_sc=False),
)
def _agg_kernel(hhf, src2, dstp, zrows, out,
                si_v, di_v, buf_v, acc_sh, sem0, sem1):
    cid = lax.axis_index("c")
    sid = lax.axis_index("s")
    r0 = sid * ROWS_T
    nck = ECHUNKS // 16
    pltpu.sync_copy(zrows, acc_sh.at[pl.ds(r0, ROWS_T)])
    pltpu.sync_copy(src2.at[cid, pl.ds(sid * nck, nck)], si_v)
    pltpu.sync_copy(dstp.at[pl.ds(sid * nck, nck)], di_v)
    plsc.subcore_barrier()

    sems = (sem0, sem1)
    pltpu.async_copy(hhf.at[si_v.at[0]], buf_v.at[0], sem0)
    pltpu.async_copy(hhf.at[si_v.at[1]], buf_v.at[1], sem1)

    def step(c, b, issue_next):
        pltpu.make_async_copy(hhf.at[si_v.at[0]], buf_v.at[b], sems[b]).wait()
        pltpu.sync_copy(buf_v.at[b], acc_sh.at[di_v.at[c]], add=True)
        if issue_next:
            pltpu.async_copy(hhf.at[si_v.at[c + 2]], buf_v.at[b], sems[b])

    def loop_body(k, carry):
        step(2 * k, 0, True)
        step(2 * k + 1, 1, True)
        return carry

    lax.fori_loop(0, nck // 2 - 1, loop_body, 0)
    step(nck - 2, 0, False)
    step(nck - 1, 1, False)
    plsc.subcore_barrier()
    pltpu.sync_copy(acc_sh.at[pl.ds(r0, ROWS_T)],
                    out.at[cid, pl.ds(r0, ROWS_T)])


def _prep_body(x_ref, ds_ref, dd_ref, hh_ref, ns_ref, nd_ref):
    dsv = ds_ref[...]
    ddv = dd_ref[...]
    deg_s = dsv[0, :, 0:1] + dsv[1, :, 0:1]
    deg_d = ddv[0, :, 0:1] + ddv[1, :, 0:1]
    ns = jnp.where(deg_s > 0, lax.rsqrt(deg_s), 0.0)
    nd = jnp.where(deg_d > 0, lax.rsqrt(deg_d), 0.0)
    h = x_ref[...] * ns
    hh_ref[0] = h[:, :DH]
    hh_ref[1] = h[:, DH:]
    ns_ref[...] = ns
    nd_ref[...] = nd


def _prep_stage(xp, deg_s, deg_d):
    return pl.pallas_call(
        _prep_body,
        grid=(NP // BLK,),
        in_specs=[
            pl.BlockSpec((BLK, D), lambda i: (i, 0)),
            pl.BlockSpec((2, BLK, 16), lambda i: (0, i, 0)),
            pl.BlockSpec((2, BLK, 16), lambda i: (0, i, 0)),
        ],
        out_specs=[
            pl.BlockSpec((2, BLK, DH), lambda i: (0, i, 0)),
            pl.BlockSpec((BLK, 1), lambda i: (i, 0)),
            pl.BlockSpec((BLK, 1), lambda i: (i, 0)),
        ],
        out_shape=[
            jax.ShapeDtypeStruct((2, NP, DH), jnp.float32),
            jax.ShapeDtypeStruct((NP, 1), jnp.float32),
            jax.ShapeDtypeStruct((NP, 1), jnp.float32),
        ],
    )(xp, deg_s, deg_d)


def _dense_body(p_ref, nd_ref, w_ref, b_ref, *rest, relu, post, split_out):
    if post:
        ns_ref = rest[0]
        rest = rest[1:]
    o_ref = rest[0]
    pv = p_ref[...]
    agg = jnp.concatenate([pv[0], pv[1]], axis=1)
    y = jnp.dot(agg * nd_ref[...], w_ref[...],
                preferred_element_type=jnp.float32) + b_ref[...]
    if relu:
        y = jnp.maximum(y, 0.0)
    if post:
        y = y * ns_ref[...]
    if split_out:
        o_ref[0] = y[:, :DH]
        o_ref[1] = y[:, DH:]
    else:
        o_ref[...] = y


def _dense_stage(p, nd, w, b, relu, post=None, split_out=False):
    args = [p, nd, w, b]
    in_specs = [
        pl.BlockSpec((2, BLK, DH), lambda i: (0, i, 0)),
        pl.BlockSpec((BLK, 1), lambda i: (i, 0)),
        pl.BlockSpec((D, D), lambda i: (0, 0)),
        pl.BlockSpec((1, D), lambda i: (0, 0)),
    ]
    if post is not None:
        args.append(post)
        in_specs.append(pl.BlockSpec((BLK, 1), lambda i: (i, 0)))
    if split_out:
        out_spec = pl.BlockSpec((2, BLK, DH), lambda i: (0, i, 0))
        out_shape = jax.ShapeDtypeStruct((2, NP, DH), jnp.float32)
    else:
        out_spec = pl.BlockSpec((BLK, D), lambda i: (i, 0))
        out_shape = jax.ShapeDtypeStruct((NP, D), jnp.float32)
    body = functools.partial(_dense_body, relu=relu, post=post is not None,
                             split_out=split_out)
    return pl.pallas_call(
        body,
        grid=(NP // BLK,),
        in_specs=in_specs,
        out_specs=out_spec,
        out_shape=out_shape,
    )(*args)


def kernel(x, edge_index, W1, b1, W2, b2):
    prev_x64 = jax.config.jax_enable_x64
    jax.config.update("jax_enable_x64", False)
    try:
        return _kernel_impl(x, edge_index, W1, b1, W2, b2)
    finally:
        jax.config.update("jax_enable_x64", prev_x64)


def _kernel_impl(x, edge_index, W1, b1, W2, b2):
    src = edge_index[0].astype(jnp.int32)
    dst = edge_index[1].astype(jnp.int32)
    x = x.astype(jnp.float32)
    W1 = W1.astype(jnp.float32)
    W2 = W2.astype(jnp.float32)
    b1 = b1.astype(jnp.float32).reshape(1, D)
    b2 = b2.astype(jnp.float32).reshape(1, D)

    pad = jnp.full((EP - E,), N, jnp.int32)
    srcp = jnp.concatenate([src, pad]).reshape(ECHUNKS, CHUNK)
    dstp = jnp.concatenate([dst, pad]).reshape(ECHUNKS, CHUNK)
    xp = jnp.zeros((NP, D), jnp.float32).at[:N].set(x)

    ones_h = jnp.ones((CHUNK, 16), jnp.float32)
    z16 = jnp.zeros((ROWS_T, 16), jnp.float32)
    zrows = jnp.zeros((ROWS_T, DH), jnp.float32)

    src2 = jnp.stack([srcp, srcp + NP])
    deg_s, deg_d = _deg_kernel(srcp, dstp, ones_h, z16)
    hh, ns, nd = _prep_stage(xp, deg_s, deg_d)
    p1 = _agg_kernel(hh.reshape(2 * NP, DH), src2, dstp, zrows)
    h1h = _dense_stage(p1, nd, W1, b1, relu=True, post=ns, split_out=True)
    p2 = _agg_kernel(h1h.reshape(2 * NP, DH), src2, dstp, zrows)
    out = _dense_stage(p2, nd, W2, b2, relu=False)
    return out[:N]
